# Initial kernel scaffold; baseline (speedup 1.0000x reference)
import jax
import jax.numpy as jnp
from jax.experimental import pallas as pl


def kernel(x, edge_index, edge_basis, W, b):
    raise NotImplementedError("write your pallas kernel here")


# speedup vs baseline: 2.8402x; 2.8402x over previous
import functools

import jax
import jax.numpy as jnp
from jax import lax
from jax.experimental import pallas as pl
from jax.experimental.pallas import tpu as pltpu
from jax.experimental.pallas import tpu_sc as plsc


def _filter_matmul(edge_basis, W, b2d):
    E, R = edge_basis.shape
    D = W.shape[0]
    BE = 3200

    def mm_kernel(a_ref, w_ref, b_ref, o_ref):
        o_ref[...] = lax.dot_general(
            a_ref[...], w_ref[...], (((1,), (1,)), ((), ())),
            preferred_element_type=jnp.float32) + b_ref[...]

    return pl.pallas_call(
        mm_kernel,
        grid=(E // BE,),
        in_specs=[
            pl.BlockSpec((BE, R), lambda i: (i, 0)),
            pl.BlockSpec((D, R), lambda i: (0, 0)),
            pl.BlockSpec((1, D), lambda i: (0, 0)),
        ],
        out_specs=pl.BlockSpec((BE, D), lambda i: (i, 0)),
        out_shape=jax.ShapeDtypeStruct((E, D), jnp.float32),
    )(edge_basis, W, b2d)


def _sc_gather_mul_scatter(x, src, dst, filt):
    N, D = x.shape
    E = src.shape[0]
    C = 128
    NCHUNK = E // C
    NW = 32
    CHUNKS_PER_W = -(-NCHUNK // NW)
    NSUB = 16
    RPT_A = 624
    RPT_LAST = N - (NSUB - 1) * RPT_A
    NVEC = D // 16

    mesh = plsc.VectorSubcoreMesh(core_axis_name="c", subcore_axis_name="s")

    @functools.partial(
        pl.kernel,
        out_type=jax.ShapeDtypeStruct((2, N, D), jnp.float32),
        mesh=mesh,
        scratch_types=[
            pltpu.VMEM((C,), jnp.int32),
            pltpu.VMEM((C,), jnp.int32),
            pltpu.VMEM((C, D), jnp.float32),
            pltpu.VMEM((C, D), jnp.float32),
            pltpu.VMEM_SHARED((N, D), jnp.float32),
            pltpu.SemaphoreType.DMA,
        ],
    )
    def k(x_hbm, src_hbm, dst_hbm, filt_hbm, out_hbm,
          src_v, dst_v, xg_v, f_v, acc_sh, sem):
        c = lax.axis_index("c")
        s = lax.axis_index("s")
        w = s * 2 + c

        zero = jnp.zeros((16,), jnp.float32)

        def zero_row(r, _):
            for kk in range(NVEC):
                xg_v[r, pl.ds(kk * 16, 16)] = zero
            return 0

        lax.fori_loop(0, C, zero_row, 0)

        @pl.when(s < NSUB - 1)
        def _():
            for p in range(RPT_A // 104):
                pltpu.sync_copy(
                    xg_v.at[pl.ds(0, 104)],
                    acc_sh.at[pl.ds(s * RPT_A + p * 104, 104)])

        @pl.when(s == NSUB - 1)
        def _():
            for p in range(RPT_LAST // C):
                pltpu.sync_copy(
                    xg_v.at[pl.ds(0, C)],
                    acc_sh.at[pl.ds((NSUB - 1) * RPT_A + p * C, C)])

        plsc.subcore_barrier()

        def body(t, _):
            j = w + t * NW

            @pl.when(j < NCHUNK)
            def _():
                base = j * C
                pltpu.sync_copy(src_hbm.at[pl.ds(base, C)], src_v)
                pltpu.sync_copy(dst_hbm.at[pl.ds(base, C)], dst_v)
                pltpu.async_copy(x_hbm.at[src_v], xg_v, sem).wait()
                pltpu.sync_copy(filt_hbm.at[pl.ds(base, C)], f_v)

                def mul_row(r, _):
                    for kk in range(NVEC):
                        sl = pl.ds(kk * 16, 16)
                        xg_v[r, sl] = xg_v[r, sl] * f_v[r, sl]
                    return 0

                lax.fori_loop(0, C, mul_row, 0)
                pltpu.sync_copy(xg_v, acc_sh.at[dst_v], add=True)

            return 0

        lax.fori_loop(0, CHUNKS_PER_W, body, 0)
        plsc.subcore_barrier()

        @pl.when(s < NSUB - 1)
        def _():
            rbase = s * RPT_A
            pltpu.sync_copy(
                acc_sh.at[pl.ds(rbase, RPT_A)],
                out_hbm.at[c, pl.ds(rbase, RPT_A)])

        @pl.when(s == NSUB - 1)
        def _():
            rbase = (NSUB - 1) * RPT_A
            pltpu.sync_copy(
                acc_sh.at[pl.ds(rbase, RPT_LAST)],
                out_hbm.at[c, pl.ds(rbase, RPT_LAST)])

    return k(x, src, dst, filt)


def _add_partials(p):
    _, N, D = p.shape
    BN = 2000

    def add_k(p_ref, o_ref):
        o_ref[...] = p_ref[0] + p_ref[1]

    return pl.pallas_call(
        add_k,
        grid=(N // BN,),
        in_specs=[pl.BlockSpec((2, BN, D), lambda i: (0, i, 0))],
        out_specs=pl.BlockSpec((BN, D), lambda i: (i, 0)),
        out_shape=jax.ShapeDtypeStruct((N, D), jnp.float32),
    )(p)


def kernel(x, edge_index, edge_basis, W, b):
    src = edge_index[0]
    dst = edge_index[1]
    filt = _filter_matmul(edge_basis, W, b.reshape(1, -1))
    partials = _sc_gather_mul_scatter(x, src, dst, filt)
    return _add_partials(partials)


# speedup vs baseline: 4.4209x; 1.5566x over previous
import functools

import jax
import jax.numpy as jnp
from jax import lax
from jax.experimental import pallas as pl
from jax.experimental.pallas import tpu as pltpu
from jax.experimental.pallas import tpu_sc as plsc


def _filter_matmul(edge_basis, W, b2d):
    E, R = edge_basis.shape
    D = W.shape[0]
    BE = 3200

    def mm_kernel(a_ref, w_ref, b_ref, o_ref):
        o_ref[...] = lax.dot_general(
            a_ref[...], w_ref[...], (((1,), (1,)), ((), ())),
            preferred_element_type=jnp.float32) + b_ref[...]

    return pl.pallas_call(
        mm_kernel,
        grid=(E // BE,),
        in_specs=[
            pl.BlockSpec((BE, R), lambda i: (i, 0)),
            pl.BlockSpec((D, R), lambda i: (0, 0)),
            pl.BlockSpec((1, D), lambda i: (0, 0)),
        ],
        out_specs=pl.BlockSpec((BE, D), lambda i: (i, 0)),
        out_shape=jax.ShapeDtypeStruct((E, D), jnp.float32),
    )(edge_basis, W, b2d)


def _sc_gather_mul_scatter(x, src, dst, filt):
    N, D = x.shape
    E = src.shape[0]
    C = 48
    NW = 32
    EPT = E // NW
    NT = (EPT // C) // 3 * 3
    TSUB = 32
    TAIL = EPT - NT * C
    NSUB = 16
    RPT_A = 624
    RPT_LAST = N - (NSUB - 1) * RPT_A
    NVEC = D // 16

    mesh = plsc.VectorSubcoreMesh(core_axis_name="c", subcore_axis_name="s")

    @functools.partial(
        pl.kernel,
        out_type=jax.ShapeDtypeStruct((2, N, D), jnp.float32),
        mesh=mesh,
        scratch_types=[
            pltpu.VMEM((EPT,), jnp.int32),
            pltpu.VMEM((C, D), jnp.float32),
            pltpu.VMEM((C, D), jnp.float32),
            pltpu.VMEM((C, D), jnp.float32),
            pltpu.VMEM((C, D), jnp.float32),
            pltpu.VMEM((C, D), jnp.float32),
            pltpu.VMEM((C, D), jnp.float32),
            pltpu.VMEM((C,), jnp.int32),
            pltpu.VMEM((C,), jnp.int32),
            pltpu.VMEM((C,), jnp.int32),
            pltpu.VMEM((TSUB,), jnp.int32),
            pltpu.VMEM_SHARED((N, D), jnp.float32),
            pltpu.SemaphoreType.DMA,
            pltpu.SemaphoreType.DMA,
            pltpu.SemaphoreType.DMA,
            pltpu.SemaphoreType.DMA,
            pltpu.SemaphoreType.DMA,
            pltpu.SemaphoreType.DMA,
            pltpu.SemaphoreType.DMA,
            pltpu.SemaphoreType.DMA,
            pltpu.SemaphoreType.DMA,
            pltpu.SemaphoreType.DMA,
            pltpu.SemaphoreType.DMA,
            pltpu.SemaphoreType.DMA,
        ],
    )
    def k(x_hbm, src_hbm, dst_hbm, filt_hbm, out_hbm,
          src_all, xg0, xg1, xg2, f0, f1, f2, d0, d1, d2, dt,
          acc_sh, gs0, gs1, gs2, fs0, fs1, fs2, ss0, ss1, ss2,
          ds0, ds1, ds2):
        xg = [xg0, xg1, xg2]
        fb = [f0, f1, f2]
        dc = [d0, d1, d2]
        gsem = [gs0, gs1, gs2]
        fsem = [fs0, fs1, fs2]
        ssem = [ss0, ss1, ss2]
        dsem = [ds0, ds1, ds2]

        c = lax.axis_index("c")
        s = lax.axis_index("s")
        w = s * 2 + c
        ebase = w * EPT

        zero = jnp.zeros((16,), jnp.float32)

        @plsc.parallel_loop(0, C)
        def _(r):
            for kk in range(NVEC):
                xg0[r, pl.ds(kk * 16, 16)] = zero

        @pl.when(s < NSUB - 1)
        def _():
            for p in range(RPT_A // C):
                pltpu.sync_copy(
                    xg0.at[pl.ds(0, C)],
                    acc_sh.at[pl.ds(s * RPT_A + p * C, C)])

        @pl.when(s == NSUB - 1)
        def _():
            lbase = (NSUB - 1) * RPT_A
            for p in range(RPT_LAST // C):
                pltpu.sync_copy(
                    xg0.at[pl.ds(0, C)],
                    acc_sh.at[pl.ds(lbase + p * C, C)])
            rem = RPT_LAST % C
            if rem:
                pltpu.sync_copy(
                    xg0.at[pl.ds(0, rem)],
                    acc_sh.at[pl.ds(lbase + (RPT_LAST // C) * C, rem)])

        plsc.subcore_barrier()

        pltpu.sync_copy(src_hbm.at[pl.ds(ebase, EPT)], src_all)

        def issue(t, b):
            pltpu.async_copy(
                x_hbm.at[src_all.at[pl.ds(t * C, C)]], xg[b], gsem[b])
            pltpu.async_copy(
                filt_hbm.at[pl.ds(ebase + t * C, C)], fb[b], fsem[b])
            pltpu.async_copy(
                dst_hbm.at[pl.ds(ebase + t * C, C)], dc[b], dsem[b])

        def wait_gather(b):
            pltpu.make_async_copy(
                x_hbm.at[src_all.at[pl.ds(0, C)]], xg[b], gsem[b]).wait()

        def wait_filt(b):
            pltpu.make_async_copy(
                filt_hbm.at[pl.ds(0, C)], fb[b], fsem[b]).wait()

        def wait_didx(b):
            pltpu.make_async_copy(
                dst_hbm.at[pl.ds(0, C)], dc[b], dsem[b]).wait()

        def wait_scat(b):
            pltpu.make_async_copy(
                filt_hbm.at[pl.ds(0, C)], xg[b], ssem[b]).wait()

        issue(0, 0)
        issue(1, 1)

        def outer(g, _):
            for b in range(3):
                t = 3 * g + b
                wait_gather(b)
                wait_filt(b)
                wait_didx(b)

                @plsc.parallel_loop(0, C)
                def _(r):
                    for kk in range(NVEC):
                        sl = pl.ds(kk * 16, 16)
                        xg[b][r, sl] = xg[b][r, sl] * fb[b][r, sl]

                pltpu.async_copy(xg[b], acc_sh.at[dc[b]], ssem[b], add=True)

                tn = t + 2
                bn = (b + 2) % 3

                @pl.when(tn < NT)
                def _():
                    @pl.when(tn >= 3)
                    def _():
                        wait_scat(bn)
                    issue(tn, bn)

            return 0

        lax.fori_loop(0, NT // 3, outer, 0)
        for b in range(3):
            wait_scat(b)

        for p in range(TAIL // TSUB):
            tb = NT * C + p * TSUB
            pltpu.async_copy(
                x_hbm.at[src_all.at[pl.ds(tb, TSUB)]],
                xg0.at[pl.ds(0, TSUB)], gs0).wait()
            pltpu.sync_copy(
                filt_hbm.at[pl.ds(ebase + tb, TSUB)], f0.at[pl.ds(0, TSUB)])
            pltpu.sync_copy(dst_hbm.at[pl.ds(ebase + tb, TSUB)], dt)

            @plsc.parallel_loop(0, TSUB)
            def _(r):
                for kk in range(NVEC):
                    sl = pl.ds(kk * 16, 16)
                    xg0[r, sl] = xg0[r, sl] * f0[r, sl]

            pltpu.sync_copy(xg0.at[pl.ds(0, TSUB)], acc_sh.at[dt], add=True)

        plsc.subcore_barrier()

        @pl.when(s < NSUB - 1)
        def _():
            rbase = s * RPT_A
            pltpu.sync_copy(
                acc_sh.at[pl.ds(rbase, RPT_A)],
                out_hbm.at[c, pl.ds(rbase, RPT_A)])

        @pl.when(s == NSUB - 1)
        def _():
            rbase = (NSUB - 1) * RPT_A
            pltpu.sync_copy(
                acc_sh.at[pl.ds(rbase, RPT_LAST)],
                out_hbm.at[c, pl.ds(rbase, RPT_LAST)])

    return k(x, src, dst, filt)


def _add_partials(p):
    _, N, D = p.shape
    BN = 2000

    def add_k(p_ref, o_ref):
        o_ref[...] = p_ref[0] + p_ref[1]

    return pl.pallas_call(
        add_k,
        grid=(N // BN,),
        in_specs=[pl.BlockSpec((2, BN, D), lambda i: (0, i, 0))],
        out_specs=pl.BlockSpec((BN, D), lambda i: (i, 0)),
        out_shape=jax.ShapeDtypeStruct((N, D), jnp.float32),
    )(p)


def kernel(x, edge_index, edge_basis, W, b):
    src = edge_index[0]
    dst = edge_index[1]
    filt = _filter_matmul(edge_basis, W, b.reshape(1, -1))
    partials = _sc_gather_mul_scatter(x, src, dst, filt)
    return _add_partials(partials)


# speedup vs baseline: 6.5229x; 1.4755x over previous
import functools

import jax
import jax.numpy as jnp
from jax import lax
from jax.experimental import pallas as pl
from jax.experimental.pallas import tpu as pltpu
from jax.experimental.pallas import tpu_sc as plsc


def _filter_matmul(edge_basis_t, W_t, b2d, e0, ne):
    R, E = edge_basis_t.shape
    D = W_t.shape[1]
    BE = 6400

    def mm_kernel(a_ref, w_ref, b_ref, o_ref):
        o_ref[...] = lax.dot_general(
            a_ref[...], w_ref[...], (((0,), (0,)), ((), ())),
            preferred_element_type=jnp.float32) + b_ref[...]

    return pl.pallas_call(
        mm_kernel,
        grid=(ne // BE,),
        in_specs=[
            pl.BlockSpec((R, BE), lambda i: (0, i + e0 // BE)),
            pl.BlockSpec((R, D), lambda i: (0, 0)),
            pl.BlockSpec((1, D), lambda i: (0, 0)),
        ],
        out_specs=pl.BlockSpec((BE, D), lambda i: (i, 0)),
        out_shape=jax.ShapeDtypeStruct((ne, D), jnp.float32),
    )(edge_basis_t, W_t, b2d)


def _sc_gather_mul_scatter(x, eidx, filt, e0, ne):
    N, D = x.shape
    E = eidx.shape[0] // 2
    C = 48
    NW = 32
    EPT = ne // NW
    NT = EPT // C
    NT3 = NT // 3 * 3
    TAIL = EPT - NT * C
    NSUB = 16
    RPT_A = 624
    RPT_LAST = N - (NSUB - 1) * RPT_A
    NVEC = D // 16

    mesh = plsc.VectorSubcoreMesh(core_axis_name="c", subcore_axis_name="s")

    @functools.partial(
        pl.kernel,
        out_type=jax.ShapeDtypeStruct((2, N, D), jnp.float32),
        mesh=mesh,
        scratch_types=[
            pltpu.VMEM((EPT,), jnp.int32),
            pltpu.VMEM((C, D), jnp.float32),
            pltpu.VMEM((C, D), jnp.float32),
            pltpu.VMEM((C, D), jnp.float32),
            pltpu.VMEM((C, D), jnp.float32),
            pltpu.VMEM((C, D), jnp.float32),
            pltpu.VMEM((C, D), jnp.float32),
            pltpu.VMEM((C,), jnp.int32),
            pltpu.VMEM((C,), jnp.int32),
            pltpu.VMEM((C,), jnp.int32),
            pltpu.VMEM((max(TAIL, 8),), jnp.int32),
            pltpu.VMEM_SHARED((N, D), jnp.float32),
            pltpu.SemaphoreType.DMA,
            pltpu.SemaphoreType.DMA,
            pltpu.SemaphoreType.DMA,
            pltpu.SemaphoreType.DMA,
            pltpu.SemaphoreType.DMA,
            pltpu.SemaphoreType.DMA,
            pltpu.SemaphoreType.DMA,
            pltpu.SemaphoreType.DMA,
            pltpu.SemaphoreType.DMA,
            pltpu.SemaphoreType.DMA,
            pltpu.SemaphoreType.DMA,
            pltpu.SemaphoreType.DMA,
        ],
    )
    def k(x_hbm, eidx_hbm, filt_hbm, out_hbm,
          src_all, xg0, xg1, xg2, f0, f1, f2, d0, d1, d2, dt,
          acc_sh, gs0, gs1, gs2, fs0, fs1, fs2, ss0, ss1, ss2,
          ds0, ds1, ds2):
        xg = [xg0, xg1, xg2]
        fb = [f0, f1, f2]
        dc = [d0, d1, d2]
        gsem = [gs0, gs1, gs2]
        fsem = [fs0, fs1, fs2]
        ssem = [ss0, ss1, ss2]
        dsem = [ds0, ds1, ds2]

        c = lax.axis_index("c")
        s = lax.axis_index("s")
        w = s * 2 + c
        ebase = w * EPT

        zero = jnp.zeros((16,), jnp.float32)

        @plsc.parallel_loop(0, C)
        def _(r):
            for kk in range(NVEC):
                xg0[r, pl.ds(kk * 16, 16)] = zero

        @pl.when(s < NSUB - 1)
        def _():
            for p in range(RPT_A // C):
                pltpu.sync_copy(
                    xg0.at[pl.ds(0, C)],
                    acc_sh.at[pl.ds(s * RPT_A + p * C, C)])

        @pl.when(s == NSUB - 1)
        def _():
            lbase = (NSUB - 1) * RPT_A
            for p in range(RPT_LAST // C):
                pltpu.sync_copy(
                    xg0.at[pl.ds(0, C)],
                    acc_sh.at[pl.ds(lbase + p * C, C)])
            rem = RPT_LAST % C
            if rem:
                pltpu.sync_copy(
                    xg0.at[pl.ds(0, rem)],
                    acc_sh.at[pl.ds(lbase + (RPT_LAST // C) * C, rem)])

        plsc.subcore_barrier()

        pltpu.sync_copy(eidx_hbm.at[pl.ds(e0 + ebase, EPT)], src_all)

        def issue(t, b):
            pltpu.async_copy(
                x_hbm.at[src_all.at[pl.ds(t * C, C)]], xg[b], gsem[b])
            pltpu.async_copy(
                filt_hbm.at[pl.ds(ebase + t * C, C)], fb[b], fsem[b])
            pltpu.async_copy(
                eidx_hbm.at[pl.ds(E + e0 + ebase + t * C, C)], dc[b], dsem[b])

        def wait_gather(b):
            pltpu.make_async_copy(
                x_hbm.at[src_all.at[pl.ds(0, C)]], xg[b], gsem[b]).wait()

        def wait_filt(b):
            pltpu.make_async_copy(
                filt_hbm.at[pl.ds(0, C)], fb[b], fsem[b]).wait()

        def wait_didx(b):
            pltpu.make_async_copy(
                eidx_hbm.at[pl.ds(0, C)], dc[b], dsem[b]).wait()

        def wait_scat(b):
            pltpu.make_async_copy(
                filt_hbm.at[pl.ds(0, C)], xg[b], ssem[b]).wait()

        def mul_chunk(xgb, fbb, rows):
            @plsc.parallel_loop(0, rows)
            def _(r):
                for kk in range(NVEC):
                    sl = pl.ds(kk * 16, 16)
                    xgb[r, sl] = xgb[r, sl] * fbb[r, sl]

        issue(0, 0)
        issue(1, 1)

        def outer(g, _):
            for b in range(3):
                t = 3 * g + b
                wait_gather(b)
                wait_filt(b)
                wait_didx(b)
                mul_chunk(xg[b], fb[b], C)
                pltpu.async_copy(xg[b], acc_sh.at[dc[b]], ssem[b], add=True)

                tn = t + 2
                bn = (b + 2) % 3

                @pl.when(tn < NT3)
                def _():
                    @pl.when(tn >= 3)
                    def _():
                        wait_scat(bn)
                    issue(tn, bn)

            return 0

        lax.fori_loop(0, NT3 // 3, outer, 0)
        for b in range(3):
            wait_scat(b)

        sizes = [C] * (NT - NT3) + ([TAIL] if TAIL else [])
        off = NT3 * C
        for sz in sizes:
            idxr = dc[0] if sz == C else dt
            pltpu.async_copy(
                x_hbm.at[src_all.at[pl.ds(off, sz)]],
                xg0.at[pl.ds(0, sz)], gs0).wait()
            pltpu.sync_copy(
                filt_hbm.at[pl.ds(ebase + off, sz)], f0.at[pl.ds(0, sz)])
            pltpu.sync_copy(eidx_hbm.at[pl.ds(E + e0 + ebase + off, sz)], idxr)
            mul_chunk(xg0, f0, sz)
            pltpu.sync_copy(xg0.at[pl.ds(0, sz)], acc_sh.at[idxr], add=True)
            off += sz

        plsc.subcore_barrier()

        @pl.when(s < NSUB - 1)
        def _():
            rbase = s * RPT_A
            pltpu.sync_copy(
                acc_sh.at[pl.ds(rbase, RPT_A)],
                out_hbm.at[c, pl.ds(rbase, RPT_A)])

        @pl.when(s == NSUB - 1)
        def _():
            rbase = (NSUB - 1) * RPT_A
            pltpu.sync_copy(
                acc_sh.at[pl.ds(rbase, RPT_LAST)],
                out_hbm.at[c, pl.ds(rbase, RPT_LAST)])

    return k(x, eidx, filt)


def _add_partials(pa, pb):
    _, N, D = pa.shape
    BN = 2000

    def add_k(pa_ref, pb_ref, o_ref):
        o_ref[...] = (pa_ref[0] + pa_ref[1]) + (pb_ref[0] + pb_ref[1])

    return pl.pallas_call(
        add_k,
        grid=(N // BN,),
        in_specs=[
            pl.BlockSpec((2, BN, D), lambda i: (0, i, 0)),
            pl.BlockSpec((2, BN, D), lambda i: (0, i, 0)),
        ],
        out_specs=pl.BlockSpec((BN, D), lambda i: (i, 0)),
        out_shape=jax.ShapeDtypeStruct((N, D), jnp.float32),
    )(pa, pb)


def kernel(x, edge_index, edge_basis, W, b):
    E = edge_index.shape[1]
    E2 = E // 2
    eidx = edge_index.reshape(-1)
    basis_t = edge_basis.T
    w_t = W.T
    b2d = b.reshape(1, -1)
    filt_a = _filter_matmul(basis_t, w_t, b2d, 0, E2)
    filt_b = _filter_matmul(basis_t, w_t, b2d, E2, E - E2)
    part_a = _sc_gather_mul_scatter(x, eidx, filt_a, 0, E2)
    part_b = _sc_gather_mul_scatter(x, eidx, filt_b, E2, E - E2)
    return _add_partials(part_a, part_b)
